# baseline (device time: 22592 ns/iter reference)
import jax
import jax.numpy as jnp
from jax import lax
from jax.experimental import pallas as pl
from jax.experimental.pallas import tpu as pltpu

K = 16


def kernel(partial, gamma):
    _, m_total, d = partial.shape
    m_half = m_total // 2
    rows = m_half // K

    def body(
        part_ref,
        gamma_ref,
        out_ref,
        peer_v,
        mine_v,
        send_q,
        recv_q,
        scale_send,
        scale_recv,
        gamma_v,
        out_v,
        local_sems,
        out_sems,
        send_sems,
        recv_sems,
    ):
        my_x = lax.axis_index("x")
        my_y = lax.axis_index("y")
        my_z = lax.axis_index("z")
        peer_x = 1 - my_x
        peer = (peer_x, my_y, my_z)

        gamma_cp = pltpu.make_async_copy(
            gamma_ref, gamma_v, local_sems.at[2 * K]
        )
        gamma_cp.start()
        peer_cp = []
        mine_cp = []
        for k in range(K):
            c = pltpu.make_async_copy(
                part_ref.at[0, pl.ds(peer_x * m_half + k * rows, rows), :],
                peer_v.at[k],
                local_sems.at[k],
            )
            c.start()
            peer_cp.append(c)
        for k in range(K):
            c = pltpu.make_async_copy(
                part_ref.at[0, pl.ds(my_x * m_half + k * rows, rows), :],
                mine_v.at[k],
                local_sems.at[K + k],
            )
            c.start()
            mine_cp.append(c)

        barrier = pltpu.get_barrier_semaphore()
        pl.semaphore_signal(
            barrier, inc=1, device_id=peer, device_id_type=pl.DeviceIdType.MESH
        )
        pl.semaphore_wait(barrier, 1)

        data_rdmas = []
        scale_rdmas = []
        for k in range(K):
            peer_cp[k].wait()
            chunk = peer_v[k]
            m = jnp.max(jnp.abs(chunk))
            qs = 127.0 / jnp.maximum(m, 1e-30)
            send_q[k] = jnp.rint(chunk * qs).astype(jnp.int8)
            scale_send[k] = jnp.full((1, 128), m * (1.0 / 127.0), jnp.float32)
            r = pltpu.make_async_remote_copy(
                src_ref=send_q.at[k],
                dst_ref=recv_q.at[k],
                send_sem=send_sems.at[k],
                recv_sem=recv_sems.at[k],
                device_id=peer,
                device_id_type=pl.DeviceIdType.MESH,
            )
            r.start()
            data_rdmas.append(r)
            s = pltpu.make_async_remote_copy(
                src_ref=scale_send.at[k],
                dst_ref=scale_recv.at[k],
                send_sem=send_sems.at[K + k],
                recv_sem=recv_sems.at[K + k],
                device_id=peer,
                device_id_type=pl.DeviceIdType.MESH,
            )
            s.start()
            scale_rdmas.append(s)

        gamma_cp.wait()
        g = gamma_v[...]

        out_cp = []
        for k in range(K):
            data_rdmas[k].wait_recv()
            scale_rdmas[k].wait_recv()
            mine_cp[k].wait()
            s = scale_recv[k, 0:1, 0:1]
            y = mine_v[k] + recv_q[k].astype(jnp.float32) * s
            inv = lax.rsqrt(jnp.mean(y * y, axis=-1, keepdims=True) + 1e-6)
            out_v[k] = (y * inv * g).astype(jnp.bfloat16)
            c = pltpu.make_async_copy(
                out_v.at[k],
                out_ref.at[pl.ds(k * rows, rows), :],
                out_sems.at[k],
            )
            c.start()
            out_cp.append(c)

        for k in range(K):
            out_cp[k].wait()
            data_rdmas[k].wait_send()
            scale_rdmas[k].wait_send()

    gamma2d = gamma.reshape(1, d)
    return pl.pallas_call(
        body,
        out_shape=jax.ShapeDtypeStruct((m_half, d), jnp.bfloat16),
        in_specs=[
            pl.BlockSpec(memory_space=pl.ANY),
            pl.BlockSpec(memory_space=pl.ANY),
        ],
        out_specs=pl.BlockSpec(memory_space=pl.ANY),
        scratch_shapes=[
            pltpu.VMEM((K, rows, d), jnp.float32),
            pltpu.VMEM((K, rows, d), jnp.float32),
            pltpu.VMEM((K, rows, d), jnp.int8),
            pltpu.VMEM((K, rows, d), jnp.int8),
            pltpu.VMEM((K, 1, 128), jnp.float32),
            pltpu.VMEM((K, 1, 128), jnp.float32),
            pltpu.VMEM((1, d), jnp.float32),
            pltpu.VMEM((K, rows, d), jnp.bfloat16),
            pltpu.SemaphoreType.DMA((2 * K + 1,)),
            pltpu.SemaphoreType.DMA((K,)),
            pltpu.SemaphoreType.DMA((2 * K,)),
            pltpu.SemaphoreType.DMA((2 * K,)),
        ],
        compiler_params=pltpu.CompilerParams(collective_id=0),
    )(partial, gamma2d)


# device time: 18934 ns/iter; 1.1932x vs baseline; 1.1932x over previous
import jax
import jax.numpy as jnp
from jax import lax
from jax.experimental import pallas as pl
from jax.experimental.pallas import tpu as pltpu

K = 16


def kernel(partial, gamma):
    _, m_total, d = partial.shape
    m_half = m_total // 2
    rows = m_half // K

    def body(
        part_ref,
        gamma_ref,
        out_ref,
        peer_v,
        mine_v,
        send_q,
        recv_q,
        scale_send,
        scale_recv,
        gamma_v,
        out_v,
        local_sems,
        out_sems,
        send_sems,
        recv_sems,
    ):
        my_x = lax.axis_index("x")
        my_y = lax.axis_index("y")
        my_z = lax.axis_index("z")
        peer_x = 1 - my_x
        peer = (peer_x, my_y, my_z)

        gamma_cp = pltpu.make_async_copy(
            gamma_ref, gamma_v, local_sems.at[2 * K]
        )
        gamma_cp.start()
        peer_cp = []
        mine_cp = []
        for k in range(K):
            c = pltpu.make_async_copy(
                part_ref.at[0, pl.ds(peer_x * m_half + k * rows, rows), :],
                peer_v.at[k],
                local_sems.at[k],
            )
            c.start()
            peer_cp.append(c)
        for k in range(K):
            c = pltpu.make_async_copy(
                part_ref.at[0, pl.ds(my_x * m_half + k * rows, rows), :],
                mine_v.at[k],
                local_sems.at[K + k],
            )
            c.start()
            mine_cp.append(c)

        barrier = pltpu.get_barrier_semaphore()
        pl.semaphore_signal(
            barrier, inc=1, device_id=peer, device_id_type=pl.DeviceIdType.MESH
        )
        pl.semaphore_wait(barrier, 1)

        data_rdmas = []
        scale_rdmas = []
        for k in range(K):
            peer_cp[k].wait()
            chunk = peer_v[k]
            m = jnp.max(jnp.abs(chunk))
            qs = 127.0 / jnp.maximum(m, 1e-30)
            send_q[k] = jnp.rint(chunk * qs).astype(jnp.int8)
            scale_send[k] = jnp.full((1, 128), m * (1.0 / 127.0), jnp.float32)
            r = pltpu.make_async_remote_copy(
                src_ref=send_q.at[k],
                dst_ref=recv_q.at[k],
                send_sem=send_sems.at[k],
                recv_sem=recv_sems.at[k],
                device_id=peer,
                device_id_type=pl.DeviceIdType.MESH,
            )
            r.start()
            data_rdmas.append(r)
            s = pltpu.make_async_remote_copy(
                src_ref=scale_send.at[k],
                dst_ref=scale_recv.at[k],
                send_sem=send_sems.at[K + k],
                recv_sem=recv_sems.at[K + k],
                device_id=peer,
                device_id_type=pl.DeviceIdType.MESH,
            )
            s.start()
            scale_rdmas.append(s)

        gamma_cp.wait()
        g = gamma_v[...]

        out_cp = []
        for k in range(K):
            data_rdmas[k].wait_recv()
            scale_rdmas[k].wait_recv()
            mine_cp[k].wait()
            s = scale_recv[k, 0:1, 0:1]
            y = mine_v[k] + recv_q[k].astype(jnp.float32) * s
            inv = lax.rsqrt(jnp.mean(y * y, axis=-1, keepdims=True) + 1e-6)
            out_v[k] = (y * inv * g).astype(jnp.bfloat16)
            c = pltpu.make_async_copy(
                out_v.at[k],
                out_ref.at[pl.ds(k * rows, rows), :],
                out_sems.at[k],
            )
            c.start()
            out_cp.append(c)

        for k in range(K):
            out_cp[k].wait()
            data_rdmas[k].wait_send()
            scale_rdmas[k].wait_send()

    gamma2d = gamma.reshape(1, d)
    partial = pltpu.with_memory_space_constraint(
        partial, pltpu.MemorySpace.HBM
    )
    gamma2d = pltpu.with_memory_space_constraint(
        gamma2d, pltpu.MemorySpace.HBM
    )
    return pl.pallas_call(
        body,
        out_shape=jax.ShapeDtypeStruct((m_half, d), jnp.bfloat16),
        in_specs=[
            pl.BlockSpec(memory_space=pl.ANY),
            pl.BlockSpec(memory_space=pl.ANY),
        ],
        out_specs=pl.BlockSpec(memory_space=pltpu.MemorySpace.HBM),
        scratch_shapes=[
            pltpu.VMEM((K, rows, d), jnp.float32),
            pltpu.VMEM((K, rows, d), jnp.float32),
            pltpu.VMEM((K, rows, d), jnp.int8),
            pltpu.VMEM((K, rows, d), jnp.int8),
            pltpu.VMEM((K, 1, 128), jnp.float32),
            pltpu.VMEM((K, 1, 128), jnp.float32),
            pltpu.VMEM((1, d), jnp.float32),
            pltpu.VMEM((K, rows, d), jnp.bfloat16),
            pltpu.SemaphoreType.DMA((2 * K + 1,)),
            pltpu.SemaphoreType.DMA((K,)),
            pltpu.SemaphoreType.DMA((2 * K,)),
            pltpu.SemaphoreType.DMA((2 * K,)),
        ],
        compiler_params=pltpu.CompilerParams(collective_id=0),
    )(partial, gamma2d)
